# baseline (speedup 1.0000x reference)
import functools
import jax
import jax.numpy as jnp
from jax import lax
from jax.experimental import pallas as pl
from jax.experimental.pallas import tpu as pltpu
from jax.experimental.pallas import tpu_sc as plsc

N = 10000
E = 320000
D = 128
G = 16
OUT = 64

NCORE = 2
NSUB = 16
NTILE = NCORE * NSUB
EPT = E // NTILE
K = 80
NCHUNK = EPT // K
RPT = N // NSUB
ZR = 25

BLK = 1000
NBLK = N // BLK

RING_R = 4
RING_I = 4
RING_2 = 2


def _prologue1_body(x_ref, w1_ref, as_ref, ad_ref, wsk_ref, bsk_ref,
                    xp_ref, als_ref, ald_ref, skip_ref):
    x = x_ref[...]
    xp = jnp.dot(x, w1_ref[...], preferred_element_type=jnp.float32)
    als = lax.dot_general(xp, as_ref[...], (((1,), (1,)), ((), ())),
                          preferred_element_type=jnp.float32)
    ald = lax.dot_general(xp, ad_ref[...], (((1,), (1,)), ((), ())),
                          preferred_element_type=jnp.float32)
    xp_ref[...] = xp
    als_ref[...] = jnp.broadcast_to(als, (als.shape[0], D))
    ald_ref[...] = jnp.broadcast_to(ald, (ald.shape[0], D))
    skip_ref[...] = jnp.dot(x, wsk_ref[...],
                            preferred_element_type=jnp.float32) + bsk_ref[...]


def _prologue1(x, W1, a_src, a_dst, W_skip, b_skip):
    return pl.pallas_call(
        _prologue1_body,
        grid=(NBLK,),
        in_specs=[
            pl.BlockSpec((BLK, D), lambda i: (i, 0)),
            pl.BlockSpec((D, D), lambda i: (0, 0)),
            pl.BlockSpec((1, D), lambda i: (0, 0)),
            pl.BlockSpec((1, D), lambda i: (0, 0)),
            pl.BlockSpec((D, D), lambda i: (0, 0)),
            pl.BlockSpec((1, D), lambda i: (0, 0)),
        ],
        out_specs=[
            pl.BlockSpec((BLK, D), lambda i: (i, 0)),
            pl.BlockSpec((BLK, D), lambda i: (i, 0)),
            pl.BlockSpec((BLK, D), lambda i: (i, 0)),
            pl.BlockSpec((BLK, D), lambda i: (i, 0)),
        ],
        out_shape=[
            jax.ShapeDtypeStruct((N, D), jnp.float32),
            jax.ShapeDtypeStruct((N, D), jnp.float32),
            jax.ShapeDtypeStruct((N, D), jnp.float32),
            jax.ShapeDtypeStruct((N, D), jnp.float32),
        ],
    )(x, W1, a_src, a_dst, W_skip, b_skip)


def _edge_body(xp, als_f, ald_f, sd4, u_out, den_out,
               sd, sidx, didx, als_v, ald_v, ee_v, rows_v, zden,
               u_sh, den_sh,
               sem_i, sem_a, sem_g, sem_s, sem_e):
    c = lax.axis_index("c")
    s = lax.axis_index("s")

    def si(g):
        return lax.rem(g, RING_I)

    def s2(g):
        return lax.rem(g, RING_2)

    def sr(g):
        return lax.rem(g, RING_R)

    def cp_sd(g):
        return pltpu.make_async_copy(sd4.at[c, s, g], sd.at[si(g)],
                                     sem_i.at[si(g)])

    def cp_als(g):
        return pltpu.make_async_copy(als_f.at[sidx.at[s2(g)]],
                                     als_v.at[s2(g)], sem_a.at[s2(g)])

    def cp_ald(g):
        return pltpu.make_async_copy(ald_f.at[didx.at[s2(g)]],
                                     ald_v.at[s2(g)], sem_a.at[s2(g)])

    def cp_rows(g):
        return pltpu.make_async_copy(xp.at[sd.at[si(g), 0]],
                                     rows_v.at[pl.ds(sr(g) * K, K)],
                                     sem_g.at[sr(g)])

    def cp_scat(g):
        return pltpu.make_async_copy(rows_v.at[pl.ds(sr(g) * K, K)],
                                     u_sh.at[sd.at[si(g), 1]],
                                     sem_s.at[si(g)])

    def cp_den(g):
        return pltpu.make_async_copy(ee_v.at[si(g)],
                                     den_sh.at[sd.at[si(g), 1]],
                                     sem_e.at[si(g)])

    def prep_idx(g):
        b = s2(g)
        for i in range(K // 16):
            sv = sd[si(g), 0, pl.ds(i * 16, 16)]
            dv = sd[si(g), 1, pl.ds(i * 16, 16)]
            sidx[b, pl.ds(i * 16, 16)] = sv * 128
            didx[b, pl.ds(i * 16, 16)] = dv * 128

    zero16 = jnp.zeros((16,), jnp.float32)
    for i in range(ZR):
        for p in range(D // 16):
            rows_v[i, pl.ds(p * 16, 16)] = zero16
    for i in range(RPT // 16 + 1):
        zden[pl.ds(i * 16, 16)] = zero16

    def zrow(g, carry):
        pltpu.sync_copy(rows_v.at[pl.ds(0, ZR)],
                        u_sh.at[pl.ds(s * RPT + g * ZR, ZR)])
        return carry
    lax.fori_loop(0, RPT // ZR, zrow, 0)

    @pl.when(s < NSUB - 1)
    def _():
        pltpu.sync_copy(zden.at[pl.ds(0, 640)],
                        den_sh.at[pl.ds(s * 640, 640)])

    @pl.when(s == NSUB - 1)
    def _():
        pltpu.sync_copy(zden.at[pl.ds(0, 400)],
                        den_sh.at[pl.ds((NSUB - 1) * 640, 400)])
    plsc.subcore_barrier()

    cp_sd(0).start()
    cp_sd(1).start()
    cp_sd(0).wait()
    prep_idx(0)
    cp_als(0).start()
    cp_ald(0).start()
    cp_rows(0).start()

    def chunk(g, carry):
        @pl.when(g >= 2)
        def _():
            cp_scat(g - 2).wait()
            cp_den(g - 2).wait()

        @pl.when(g + 2 < NCHUNK)
        def _():
            cp_sd(g + 2).start()

        @pl.when(g + 1 < NCHUNK)
        def _():
            cp_sd(g + 1).wait()
            prep_idx(g + 1)
            cp_als(g + 1).start()
            cp_ald(g + 1).start()
            cp_rows(g + 1).start()

        cp_als(g).wait()
        cp_ald(g).wait()
        cp_rows(g).wait()

        b2 = s2(g)
        bi = si(g)
        for i in range(K // 16):
            t = (als_v[b2, pl.ds(i * 16, 16)] + ald_v[b2, pl.ds(i * 16, 16)])
            e = jnp.where(t >= 0.0, t, 0.2 * t)
            ee_v[bi, pl.ds(i * 16, 16)] = jnp.exp(e)

        cp_den(g).start(add=True)

        base = sr(g) * K

        def srow(j4, cc):
            j = j4 * 4
            for u in range(4):
                eej = plsc.load_gather(
                    ee_v, [jnp.full((16,), bi, jnp.int32),
                           jnp.full((16,), j + u, jnp.int32)])
                r = base + j + u
                for p in range(D // 16):
                    rows_v[r, pl.ds(p * 16, 16)] = (
                        rows_v[r, pl.ds(p * 16, 16)] * eej)
            return cc
        lax.fori_loop(0, K // 4, srow, 0)

        cp_scat(g).start(add=True)
        return carry
    lax.fori_loop(0, NCHUNK, chunk, 0)
    cp_scat(NCHUNK - 2).wait()
    cp_den(NCHUNK - 2).wait()
    cp_scat(NCHUNK - 1).wait()
    cp_den(NCHUNK - 1).wait()
    plsc.subcore_barrier()

    pltpu.sync_copy(u_sh.at[pl.ds(s * RPT, RPT)],
                    u_out.at[c, pl.ds(s * RPT, RPT)])

    @pl.when(s < NSUB - 1)
    def _():
        pltpu.sync_copy(den_sh.at[pl.ds(s * 640, 640)],
                        den_out.at[c, pl.ds(s * 640, 640)])

    @pl.when(s == NSUB - 1)
    def _():
        pltpu.sync_copy(den_sh.at[pl.ds((NSUB - 1) * 640, 400)],
                        den_out.at[c, pl.ds((NSUB - 1) * 640, 400)])


@functools.lru_cache(maxsize=1)
def _edge_kernel():
    return pl.kernel(
        _edge_body,
        out_type=(jax.ShapeDtypeStruct((NCORE, N, D), jnp.float32),
                  jax.ShapeDtypeStruct((NCORE, N), jnp.float32)),
        mesh=plsc.VectorSubcoreMesh(core_axis_name="c", subcore_axis_name="s"),
        compiler_params=pltpu.CompilerParams(use_tc_tiling_on_sc=False,
                                             needs_layout_passes=False),
        scratch_types=[
            pltpu.VMEM((RING_I, 2, K), jnp.int32),
            pltpu.VMEM((RING_2, K), jnp.int32),
            pltpu.VMEM((RING_2, K), jnp.int32),
            pltpu.VMEM((RING_2, K), jnp.float32),
            pltpu.VMEM((RING_2, K), jnp.float32),
            pltpu.VMEM((RING_I, K), jnp.float32),
            pltpu.VMEM((RING_R * K, D), jnp.float32),
            pltpu.VMEM((RPT + 15, ), jnp.float32),
            pltpu.VMEM_SHARED((N, D), jnp.float32),
            pltpu.VMEM_SHARED((N,), jnp.float32),
            pltpu.SemaphoreType.DMA((RING_I,)),
            pltpu.SemaphoreType.DMA((RING_2,)),
            pltpu.SemaphoreType.DMA((RING_R,)),
            pltpu.SemaphoreType.DMA((RING_I,)),
            pltpu.SemaphoreType.DMA((RING_I,)),
        ],
    )


def _mid_body(u_ref, den_ref, skip_ref, b1_ref, g1_ref, bb1_ref,
              w2_ref, as_ref, ad_ref,
              xp_ref, als_ref, ald_ref, x1_ref):
    u = u_ref[0] + u_ref[1]
    gat = u / (den_ref[...] + 1e-16) + b1_ref[...]
    m = jnp.mean(gat, axis=-1, keepdims=True)
    v = jnp.mean((gat - m) ** 2, axis=-1, keepdims=True)
    ln = (gat - m) / jnp.sqrt(v + 1e-5) * g1_ref[...] + bb1_ref[...]
    elu = jnp.where(ln > 0, ln, jnp.exp(jnp.minimum(ln, 0.0)) - 1.0)
    x1 = elu + skip_ref[...]
    x1_ref[...] = x1
    xp = jnp.dot(x1, w2_ref[...], preferred_element_type=jnp.float32)
    als = lax.dot_general(xp, as_ref[...], (((1,), (1,)), ((), ())),
                          preferred_element_type=jnp.float32)
    ald = lax.dot_general(xp, ad_ref[...], (((1,), (1,)), ((), ())),
                          preferred_element_type=jnp.float32)
    xp_ref[...] = xp
    als_ref[...] = jnp.broadcast_to(als, (als.shape[0], D))
    ald_ref[...] = jnp.broadcast_to(ald, (ald.shape[0], D))


def _mid(u, den, skip, b1, ln1_g, ln1_b, W2, a_src, a_dst):
    return pl.pallas_call(
        _mid_body,
        grid=(NBLK,),
        in_specs=[
            pl.BlockSpec((2, BLK, D), lambda i: (0, i, 0)),
            pl.BlockSpec((BLK, 1), lambda i: (i, 0)),
            pl.BlockSpec((BLK, D), lambda i: (i, 0)),
            pl.BlockSpec((1, D), lambda i: (0, 0)),
            pl.BlockSpec((1, D), lambda i: (0, 0)),
            pl.BlockSpec((1, D), lambda i: (0, 0)),
            pl.BlockSpec((D, D), lambda i: (0, 0)),
            pl.BlockSpec((1, D), lambda i: (0, 0)),
            pl.BlockSpec((1, D), lambda i: (0, 0)),
        ],
        out_specs=[
            pl.BlockSpec((BLK, D), lambda i: (i, 0)),
            pl.BlockSpec((BLK, D), lambda i: (i, 0)),
            pl.BlockSpec((BLK, D), lambda i: (i, 0)),
            pl.BlockSpec((BLK, D), lambda i: (i, 0)),
        ],
        out_shape=[
            jax.ShapeDtypeStruct((N, D), jnp.float32),
            jax.ShapeDtypeStruct((N, D), jnp.float32),
            jax.ShapeDtypeStruct((N, D), jnp.float32),
            jax.ShapeDtypeStruct((N, D), jnp.float32),
        ],
    )(u, den, skip, b1, ln1_g, ln1_b, W2, a_src, a_dst)


def _final_body(u_ref, den_ref, x1_ref, b2_ref, g2_ref, bb2_ref, batch_ref,
                wfc_ref, bfc_ref, bng_ref, bnb_ref,
                out_ref, acc_sum, acc_cnt):
    i = pl.program_id(0)
    u = u_ref[0] + u_ref[1]
    gat = u / (den_ref[...] + 1e-16) + b2_ref[...]
    pre = gat + x1_ref[...]
    m = jnp.mean(pre, axis=-1, keepdims=True)
    v = jnp.mean((pre - m) ** 2, axis=-1, keepdims=True)
    x2 = (pre - m) / jnp.sqrt(v + 1e-5) * g2_ref[...] + bb2_ref[...]
    emb = jnp.where(x2 > 0, x2, jnp.exp(jnp.minimum(x2, 0.0)) - 1.0)

    batch = batch_ref[...]
    gids = lax.broadcasted_iota(jnp.int32, (1, G), 1)
    mask = (batch == gids).astype(jnp.float32)
    part_sum = lax.dot_general(mask, emb, (((0,), (0,)), ((), ())),
                               preferred_element_type=jnp.float32)
    ones_blk = jnp.ones((emb.shape[0], D), jnp.float32)
    part_cnt = lax.dot_general(mask, ones_blk, (((0,), (0,)), ((), ())),
                               preferred_element_type=jnp.float32)

    @pl.when(i == 0)
    def _():
        acc_sum[...] = jnp.zeros_like(acc_sum)
        acc_cnt[...] = jnp.zeros_like(acc_cnt)

    acc_sum[...] += part_sum
    acc_cnt[...] += part_cnt

    @pl.when(i == NBLK - 1)
    def _():
        graph_emb = acc_sum[...] / jnp.maximum(acc_cnt[...], 1.0)
        logits = jnp.dot(graph_emb, wfc_ref[...],
                         preferred_element_type=jnp.float32) + bfc_ref[...]
        out_ref[...] = logits / jnp.sqrt(1.0 + 1e-5) * bng_ref[...] + bnb_ref[...]


def _final(u, den, x1, b2, ln2_g, ln2_b, batch, W_fc, b_fc, bn_g, bn_b):
    return pl.pallas_call(
        _final_body,
        grid=(NBLK,),
        in_specs=[
            pl.BlockSpec((2, BLK, D), lambda i: (0, i, 0)),
            pl.BlockSpec((BLK, 1), lambda i: (i, 0)),
            pl.BlockSpec((BLK, D), lambda i: (i, 0)),
            pl.BlockSpec((1, D), lambda i: (0, 0)),
            pl.BlockSpec((1, D), lambda i: (0, 0)),
            pl.BlockSpec((1, D), lambda i: (0, 0)),
            pl.BlockSpec((BLK, 1), lambda i: (i, 0)),
            pl.BlockSpec((D, OUT), lambda i: (0, 0)),
            pl.BlockSpec((1, OUT), lambda i: (0, 0)),
            pl.BlockSpec((1, OUT), lambda i: (0, 0)),
            pl.BlockSpec((1, OUT), lambda i: (0, 0)),
        ],
        out_specs=pl.BlockSpec((G, OUT), lambda i: (0, 0)),
        out_shape=jax.ShapeDtypeStruct((G, OUT), jnp.float32),
        scratch_shapes=[
            pltpu.VMEM((G, D), jnp.float32),
            pltpu.VMEM((G, D), jnp.float32),
        ],
    )(u, den, x1, b2, ln2_g, ln2_b, batch, W_fc, b_fc, bn_g, bn_b)


def kernel(x, edge_index, batch, W1, a1_src, a1_dst, b1, ln1_g, ln1_b,
           W_skip, b_skip, W2, a2_src, a2_dst, b2, ln2_g, ln2_b,
           W_fc, b_fc, bn_g, bn_b):
    src4 = edge_index[0].reshape(NCORE, NSUB, NCHUNK, K)
    dst4 = edge_index[1].reshape(NCORE, NSUB, NCHUNK, K)
    sd4 = jnp.stack([src4, dst4], axis=3)

    xp1, als1, ald1, skip = _prologue1(
        x, W1, a1_src, a1_dst, W_skip, b_skip.reshape(1, D))
    U1, den1 = _edge_kernel()(xp1, als1.reshape(N * D), ald1.reshape(N * D),
                              sd4)
    den1n = (den1[0] + den1[1]).reshape(N, 1)
    xp2, als2, ald2, x1 = _mid(
        U1, den1n, skip, b1.reshape(1, D), ln1_g.reshape(1, D),
        ln1_b.reshape(1, D), W2, a2_src, a2_dst)
    U2, den2 = _edge_kernel()(xp2, als2.reshape(N * D), ald2.reshape(N * D),
                              sd4)
    den2n = (den2[0] + den2[1]).reshape(N, 1)
    logits = _final(
        U2, den2n, x1, b2.reshape(1, D), ln2_g.reshape(1, D),
        ln2_b.reshape(1, D), batch.reshape(N, 1).astype(jnp.int32),
        W_fc, b_fc.reshape(1, OUT), bn_g.reshape(1, OUT), bn_b.reshape(1, OUT))
    return logits


# speedup vs baseline: 1.0201x; 1.0201x over previous
import functools
import jax
import jax.numpy as jnp
from jax import lax
from jax.experimental import pallas as pl
from jax.experimental.pallas import tpu as pltpu
from jax.experimental.pallas import tpu_sc as plsc

N = 10000
E = 320000
D = 128
G = 16
OUT = 64

NCORE = 2
NSUB = 16
NTILE = NCORE * NSUB
EPT = E // NTILE
K = 80
NCHUNK = EPT // K
RPT = N // NSUB
ZR = 25

BLK = 1000
NBLK = N // BLK

RING_R = 4
RING_I = 4
RING_2 = 2


def _prologue1_body(x_ref, w1_ref, as_ref, ad_ref, wsk_ref, bsk_ref,
                    xp_ref, als_ref, ald_ref, skip_ref):
    x = x_ref[...]
    xp = jnp.dot(x, w1_ref[...], preferred_element_type=jnp.float32)
    als = lax.dot_general(xp, as_ref[...], (((1,), (1,)), ((), ())),
                          preferred_element_type=jnp.float32)
    ald = lax.dot_general(xp, ad_ref[...], (((1,), (1,)), ((), ())),
                          preferred_element_type=jnp.float32)
    xp_ref[...] = xp.astype(jnp.bfloat16)
    als_ref[...] = jnp.broadcast_to(als, (als.shape[0], D))
    ald_ref[...] = jnp.broadcast_to(ald, (ald.shape[0], D))
    skip_ref[...] = jnp.dot(x, wsk_ref[...],
                            preferred_element_type=jnp.float32) + bsk_ref[...]


def _prologue1(x, W1, a_src, a_dst, W_skip, b_skip):
    return pl.pallas_call(
        _prologue1_body,
        grid=(NBLK,),
        in_specs=[
            pl.BlockSpec((BLK, D), lambda i: (i, 0)),
            pl.BlockSpec((D, D), lambda i: (0, 0)),
            pl.BlockSpec((1, D), lambda i: (0, 0)),
            pl.BlockSpec((1, D), lambda i: (0, 0)),
            pl.BlockSpec((D, D), lambda i: (0, 0)),
            pl.BlockSpec((1, D), lambda i: (0, 0)),
        ],
        out_specs=[
            pl.BlockSpec((BLK, D), lambda i: (i, 0)),
            pl.BlockSpec((BLK, D), lambda i: (i, 0)),
            pl.BlockSpec((BLK, D), lambda i: (i, 0)),
            pl.BlockSpec((BLK, D), lambda i: (i, 0)),
        ],
        out_shape=[
            jax.ShapeDtypeStruct((N, D), jnp.bfloat16),
            jax.ShapeDtypeStruct((N, D), jnp.float32),
            jax.ShapeDtypeStruct((N, D), jnp.float32),
            jax.ShapeDtypeStruct((N, D), jnp.float32),
        ],
    )(x, W1, a_src, a_dst, W_skip, b_skip)


def _edge_body(xp, als_f, ald_f, sd4, u_out, den_out,
               sd, sidx, didx, als_v, ald_v, ee_v, rows_v, zden,
               u_sh, den_sh,
               sem_i, sem_a, sem_g, sem_s, sem_e):
    c = lax.axis_index("c")
    s = lax.axis_index("s")

    def si(g):
        return lax.rem(g, RING_I)

    def s2(g):
        return lax.rem(g, RING_2)

    def sr(g):
        return lax.rem(g, RING_R)

    def cp_sd(g):
        return pltpu.make_async_copy(sd4.at[c, s, g], sd.at[si(g)],
                                     sem_i.at[si(g)])

    def cp_als(g):
        return pltpu.make_async_copy(als_f.at[sidx.at[s2(g)]],
                                     als_v.at[s2(g)], sem_a.at[s2(g)])

    def cp_ald(g):
        return pltpu.make_async_copy(ald_f.at[didx.at[s2(g)]],
                                     ald_v.at[s2(g)], sem_a.at[s2(g)])

    def cp_rows(g):
        return pltpu.make_async_copy(xp.at[sd.at[si(g), 0]],
                                     rows_v.at[pl.ds(sr(g) * K, K)],
                                     sem_g.at[sr(g)])

    def cp_scat(g):
        return pltpu.make_async_copy(rows_v.at[pl.ds(sr(g) * K, K)],
                                     u_sh.at[sd.at[si(g), 1]],
                                     sem_s.at[si(g)])

    def cp_den(g):
        return pltpu.make_async_copy(ee_v.at[si(g)],
                                     den_sh.at[sd.at[si(g), 1]],
                                     sem_e.at[si(g)])

    def prep_idx(g):
        b = s2(g)
        for i in range(K // 16):
            sv = sd[si(g), 0, pl.ds(i * 16, 16)]
            dv = sd[si(g), 1, pl.ds(i * 16, 16)]
            sidx[b, pl.ds(i * 16, 16)] = sv * 128
            didx[b, pl.ds(i * 16, 16)] = dv * 128

    zero16 = jnp.zeros((16,), jnp.float32)
    zero32 = jnp.zeros((32,), jnp.bfloat16)
    for i in range(ZR):
        for p in range(D // 32):
            rows_v[i, pl.ds(p * 32, 32)] = zero32
    for i in range(RPT // 16 + 1):
        zden[pl.ds(i * 16, 16)] = zero16

    def zrow(g, carry):
        pltpu.sync_copy(rows_v.at[pl.ds(0, ZR)],
                        u_sh.at[pl.ds(s * RPT + g * ZR, ZR)])
        return carry
    lax.fori_loop(0, RPT // ZR, zrow, 0)

    @pl.when(s < NSUB - 1)
    def _():
        pltpu.sync_copy(zden.at[pl.ds(0, 640)],
                        den_sh.at[pl.ds(s * 640, 640)])

    @pl.when(s == NSUB - 1)
    def _():
        pltpu.sync_copy(zden.at[pl.ds(0, 400)],
                        den_sh.at[pl.ds((NSUB - 1) * 640, 400)])
    plsc.subcore_barrier()

    cp_sd(0).start()
    cp_sd(1).start()
    cp_sd(0).wait()
    prep_idx(0)
    cp_als(0).start()
    cp_ald(0).start()
    cp_rows(0).start()

    def chunk(g, carry):
        @pl.when(g >= 2)
        def _():
            cp_scat(g - 2).wait()
            cp_den(g - 2).wait()

        @pl.when(g + 2 < NCHUNK)
        def _():
            cp_sd(g + 2).start()

        @pl.when(g + 1 < NCHUNK)
        def _():
            cp_sd(g + 1).wait()
            prep_idx(g + 1)
            cp_als(g + 1).start()
            cp_ald(g + 1).start()
            cp_rows(g + 1).start()

        cp_als(g).wait()
        cp_ald(g).wait()
        cp_rows(g).wait()

        b2 = s2(g)
        bi = si(g)
        for i in range(K // 16):
            t = (als_v[b2, pl.ds(i * 16, 16)] + ald_v[b2, pl.ds(i * 16, 16)])
            e = jnp.where(t >= 0.0, t, 0.2 * t)
            ee_v[bi, pl.ds(i * 16, 16)] = jnp.exp(e)

        cp_den(g).start(add=True)

        base = sr(g) * K

        def srow(j4, cc):
            j = j4 * 4
            for u in range(4):
                eej = plsc.load_gather(
                    ee_v, [jnp.full((16,), bi, jnp.int32),
                           jnp.full((16,), j + u, jnp.int32)])
                eeb = plsc.pack(eej, eej, format=plsc.PackFormat.INTERLEAVED)
                r = base + j + u
                for p in range(D // 32):
                    rows_v[r, pl.ds(p * 32, 32)] = (
                        rows_v[r, pl.ds(p * 32, 32)] * eeb)
            return cc
        lax.fori_loop(0, K // 4, srow, 0)

        cp_scat(g).start(add=True)
        return carry
    lax.fori_loop(0, NCHUNK, chunk, 0)
    cp_scat(NCHUNK - 2).wait()
    cp_den(NCHUNK - 2).wait()
    cp_scat(NCHUNK - 1).wait()
    cp_den(NCHUNK - 1).wait()
    plsc.subcore_barrier()

    pltpu.sync_copy(u_sh.at[pl.ds(s * RPT, RPT)],
                    u_out.at[c, pl.ds(s * RPT, RPT)])

    @pl.when(s < NSUB - 1)
    def _():
        pltpu.sync_copy(den_sh.at[pl.ds(s * 640, 640)],
                        den_out.at[c, pl.ds(s * 640, 640)])

    @pl.when(s == NSUB - 1)
    def _():
        pltpu.sync_copy(den_sh.at[pl.ds((NSUB - 1) * 640, 400)],
                        den_out.at[c, pl.ds((NSUB - 1) * 640, 400)])


@functools.lru_cache(maxsize=1)
def _edge_kernel():
    return pl.kernel(
        _edge_body,
        out_type=(jax.ShapeDtypeStruct((NCORE, N, D), jnp.bfloat16),
                  jax.ShapeDtypeStruct((NCORE, N), jnp.float32)),
        mesh=plsc.VectorSubcoreMesh(core_axis_name="c", subcore_axis_name="s"),
        compiler_params=pltpu.CompilerParams(use_tc_tiling_on_sc=False,
                                             needs_layout_passes=False),
        scratch_types=[
            pltpu.VMEM((RING_I, 2, K), jnp.int32),
            pltpu.VMEM((RING_2, K), jnp.int32),
            pltpu.VMEM((RING_2, K), jnp.int32),
            pltpu.VMEM((RING_2, K), jnp.float32),
            pltpu.VMEM((RING_2, K), jnp.float32),
            pltpu.VMEM((RING_I, K), jnp.float32),
            pltpu.VMEM((RING_R * K, D), jnp.bfloat16),
            pltpu.VMEM((RPT + 15, ), jnp.float32),
            pltpu.VMEM_SHARED((N, D), jnp.bfloat16),
            pltpu.VMEM_SHARED((N,), jnp.float32),
            pltpu.SemaphoreType.DMA((RING_I,)),
            pltpu.SemaphoreType.DMA((RING_2,)),
            pltpu.SemaphoreType.DMA((RING_R,)),
            pltpu.SemaphoreType.DMA((RING_I,)),
            pltpu.SemaphoreType.DMA((RING_I,)),
        ],
    )


def _mid_body(u_ref, den_ref, skip_ref, b1_ref, g1_ref, bb1_ref,
              w2_ref, as_ref, ad_ref,
              xp_ref, als_ref, ald_ref, x1_ref):
    u = u_ref[0].astype(jnp.float32) + u_ref[1].astype(jnp.float32)
    gat = u / (den_ref[...] + 1e-16) + b1_ref[...]
    m = jnp.mean(gat, axis=-1, keepdims=True)
    v = jnp.mean((gat - m) ** 2, axis=-1, keepdims=True)
    ln = (gat - m) / jnp.sqrt(v + 1e-5) * g1_ref[...] + bb1_ref[...]
    elu = jnp.where(ln > 0, ln, jnp.exp(jnp.minimum(ln, 0.0)) - 1.0)
    x1 = elu + skip_ref[...]
    x1_ref[...] = x1
    xp = jnp.dot(x1, w2_ref[...], preferred_element_type=jnp.float32)
    als = lax.dot_general(xp, as_ref[...], (((1,), (1,)), ((), ())),
                          preferred_element_type=jnp.float32)
    ald = lax.dot_general(xp, ad_ref[...], (((1,), (1,)), ((), ())),
                          preferred_element_type=jnp.float32)
    xp_ref[...] = xp.astype(jnp.bfloat16)
    als_ref[...] = jnp.broadcast_to(als, (als.shape[0], D))
    ald_ref[...] = jnp.broadcast_to(ald, (ald.shape[0], D))


def _mid(u, den, skip, b1, ln1_g, ln1_b, W2, a_src, a_dst):
    return pl.pallas_call(
        _mid_body,
        grid=(NBLK,),
        in_specs=[
            pl.BlockSpec((2, BLK, D), lambda i: (0, i, 0)),
            pl.BlockSpec((BLK, 1), lambda i: (i, 0)),
            pl.BlockSpec((BLK, D), lambda i: (i, 0)),
            pl.BlockSpec((1, D), lambda i: (0, 0)),
            pl.BlockSpec((1, D), lambda i: (0, 0)),
            pl.BlockSpec((1, D), lambda i: (0, 0)),
            pl.BlockSpec((D, D), lambda i: (0, 0)),
            pl.BlockSpec((1, D), lambda i: (0, 0)),
            pl.BlockSpec((1, D), lambda i: (0, 0)),
        ],
        out_specs=[
            pl.BlockSpec((BLK, D), lambda i: (i, 0)),
            pl.BlockSpec((BLK, D), lambda i: (i, 0)),
            pl.BlockSpec((BLK, D), lambda i: (i, 0)),
            pl.BlockSpec((BLK, D), lambda i: (i, 0)),
        ],
        out_shape=[
            jax.ShapeDtypeStruct((N, D), jnp.bfloat16),
            jax.ShapeDtypeStruct((N, D), jnp.float32),
            jax.ShapeDtypeStruct((N, D), jnp.float32),
            jax.ShapeDtypeStruct((N, D), jnp.float32),
        ],
    )(u, den, skip, b1, ln1_g, ln1_b, W2, a_src, a_dst)


def _final_body(u_ref, den_ref, x1_ref, b2_ref, g2_ref, bb2_ref, batch_ref,
                wfc_ref, bfc_ref, bng_ref, bnb_ref,
                out_ref, acc_sum, acc_cnt):
    i = pl.program_id(0)
    u = u_ref[0].astype(jnp.float32) + u_ref[1].astype(jnp.float32)
    gat = u / (den_ref[...] + 1e-16) + b2_ref[...]
    pre = gat + x1_ref[...]
    m = jnp.mean(pre, axis=-1, keepdims=True)
    v = jnp.mean((pre - m) ** 2, axis=-1, keepdims=True)
    x2 = (pre - m) / jnp.sqrt(v + 1e-5) * g2_ref[...] + bb2_ref[...]
    emb = jnp.where(x2 > 0, x2, jnp.exp(jnp.minimum(x2, 0.0)) - 1.0)

    batch = batch_ref[...]
    gids = lax.broadcasted_iota(jnp.int32, (1, G), 1)
    mask = (batch == gids).astype(jnp.float32)
    part_sum = lax.dot_general(mask, emb, (((0,), (0,)), ((), ())),
                               preferred_element_type=jnp.float32)
    ones_blk = jnp.ones((emb.shape[0], D), jnp.float32)
    part_cnt = lax.dot_general(mask, ones_blk, (((0,), (0,)), ((), ())),
                               preferred_element_type=jnp.float32)

    @pl.when(i == 0)
    def _():
        acc_sum[...] = jnp.zeros_like(acc_sum)
        acc_cnt[...] = jnp.zeros_like(acc_cnt)

    acc_sum[...] += part_sum
    acc_cnt[...] += part_cnt

    @pl.when(i == NBLK - 1)
    def _():
        graph_emb = acc_sum[...] / jnp.maximum(acc_cnt[...], 1.0)
        logits = jnp.dot(graph_emb, wfc_ref[...],
                         preferred_element_type=jnp.float32) + bfc_ref[...]
        out_ref[...] = logits / jnp.sqrt(1.0 + 1e-5) * bng_ref[...] + bnb_ref[...]


def _final(u, den, x1, b2, ln2_g, ln2_b, batch, W_fc, b_fc, bn_g, bn_b):
    return pl.pallas_call(
        _final_body,
        grid=(NBLK,),
        in_specs=[
            pl.BlockSpec((2, BLK, D), lambda i: (0, i, 0)),
            pl.BlockSpec((BLK, 1), lambda i: (i, 0)),
            pl.BlockSpec((BLK, D), lambda i: (i, 0)),
            pl.BlockSpec((1, D), lambda i: (0, 0)),
            pl.BlockSpec((1, D), lambda i: (0, 0)),
            pl.BlockSpec((1, D), lambda i: (0, 0)),
            pl.BlockSpec((BLK, 1), lambda i: (i, 0)),
            pl.BlockSpec((D, OUT), lambda i: (0, 0)),
            pl.BlockSpec((1, OUT), lambda i: (0, 0)),
            pl.BlockSpec((1, OUT), lambda i: (0, 0)),
            pl.BlockSpec((1, OUT), lambda i: (0, 0)),
        ],
        out_specs=pl.BlockSpec((G, OUT), lambda i: (0, 0)),
        out_shape=jax.ShapeDtypeStruct((G, OUT), jnp.float32),
        scratch_shapes=[
            pltpu.VMEM((G, D), jnp.float32),
            pltpu.VMEM((G, D), jnp.float32),
        ],
    )(u, den, x1, b2, ln2_g, ln2_b, batch, W_fc, b_fc, bn_g, bn_b)


def kernel(x, edge_index, batch, W1, a1_src, a1_dst, b1, ln1_g, ln1_b,
           W_skip, b_skip, W2, a2_src, a2_dst, b2, ln2_g, ln2_b,
           W_fc, b_fc, bn_g, bn_b):
    src4 = edge_index[0].reshape(NCORE, NSUB, NCHUNK, K)
    dst4 = edge_index[1].reshape(NCORE, NSUB, NCHUNK, K)
    sd4 = jnp.stack([src4, dst4], axis=3)

    xp1, als1, ald1, skip = _prologue1(
        x, W1, a1_src, a1_dst, W_skip, b_skip.reshape(1, D))
    U1, den1 = _edge_kernel()(xp1, als1.reshape(N * D), ald1.reshape(N * D),
                              sd4)
    den1n = (den1[0] + den1[1]).reshape(N, 1)
    xp2, als2, ald2, x1 = _mid(
        U1, den1n, skip, b1.reshape(1, D), ln1_g.reshape(1, D),
        ln1_b.reshape(1, D), W2, a2_src, a2_dst)
    U2, den2 = _edge_kernel()(xp2, als2.reshape(N * D), ald2.reshape(N * D),
                              sd4)
    den2n = (den2[0] + den2[1]).reshape(N, 1)
    logits = _final(
        U2, den2n, x1, b2.reshape(1, D), ln2_g.reshape(1, D),
        ln2_b.reshape(1, D), batch.reshape(N, 1).astype(jnp.int32),
        W_fc, b_fc.reshape(1, OUT), bn_g.reshape(1, OUT), bn_b.reshape(1, OUT))
    return logits


# speedup vs baseline: 1.0617x; 1.0408x over previous
import functools
import jax
import jax.numpy as jnp
from jax import lax
from jax.experimental import pallas as pl
from jax.experimental.pallas import tpu as pltpu
from jax.experimental.pallas import tpu_sc as plsc

N = 10000
E = 320000
D = 128
G = 16
OUT = 64

NCORE = 2
NSUB = 16
NTILE = NCORE * NSUB
EPT = E // NTILE
K = 80
NCHUNK = EPT // K
RPT = N // NSUB
ZR = 25

BLK = 1000
NBLK = N // BLK

RING_R = 5
RING_I = 6
RING_2 = 4


def _prologue1_body(x_ref, w1_ref, as_ref, ad_ref, wsk_ref, bsk_ref,
                    xp_ref, als_ref, ald_ref, skip_ref):
    x = x_ref[...]
    xp = jnp.dot(x, w1_ref[...], preferred_element_type=jnp.float32)
    als = lax.dot_general(xp, as_ref[...], (((1,), (1,)), ((), ())),
                          preferred_element_type=jnp.float32)
    ald = lax.dot_general(xp, ad_ref[...], (((1,), (1,)), ((), ())),
                          preferred_element_type=jnp.float32)
    xp_ref[...] = xp.astype(jnp.bfloat16)
    als_ref[...] = jnp.broadcast_to(als, (als.shape[0], D))
    ald_ref[...] = jnp.broadcast_to(ald, (ald.shape[0], D))
    skip_ref[...] = jnp.dot(x, wsk_ref[...],
                            preferred_element_type=jnp.float32) + bsk_ref[...]


def _prologue1(x, W1, a_src, a_dst, W_skip, b_skip):
    return pl.pallas_call(
        _prologue1_body,
        grid=(NBLK,),
        in_specs=[
            pl.BlockSpec((BLK, D), lambda i: (i, 0)),
            pl.BlockSpec((D, D), lambda i: (0, 0)),
            pl.BlockSpec((1, D), lambda i: (0, 0)),
            pl.BlockSpec((1, D), lambda i: (0, 0)),
            pl.BlockSpec((D, D), lambda i: (0, 0)),
            pl.BlockSpec((1, D), lambda i: (0, 0)),
        ],
        out_specs=[
            pl.BlockSpec((BLK, D), lambda i: (i, 0)),
            pl.BlockSpec((BLK, D), lambda i: (i, 0)),
            pl.BlockSpec((BLK, D), lambda i: (i, 0)),
            pl.BlockSpec((BLK, D), lambda i: (i, 0)),
        ],
        out_shape=[
            jax.ShapeDtypeStruct((N, D), jnp.bfloat16),
            jax.ShapeDtypeStruct((N, D), jnp.float32),
            jax.ShapeDtypeStruct((N, D), jnp.float32),
            jax.ShapeDtypeStruct((N, D), jnp.float32),
        ],
    )(x, W1, a_src, a_dst, W_skip, b_skip)


def _edge_body(xp, als_f, ald_f, sd4, u_out, den_out,
               sd, sidx, didx, als_v, ald_v, ee_v, rows_v, zden,
               u_sh, den_sh,
               sem_i, sem_a, sem_g, sem_s, sem_e):
    c = lax.axis_index("c")
    s = lax.axis_index("s")

    def si(g):
        return lax.rem(g, RING_I)

    def s2(g):
        return lax.rem(g, RING_2)

    def sr(g):
        return lax.rem(g, RING_R)

    def cp_sd(g):
        return pltpu.make_async_copy(sd4.at[c, s, g], sd.at[si(g)],
                                     sem_i.at[si(g)])

    def cp_als(g):
        return pltpu.make_async_copy(als_f.at[sidx.at[s2(g)]],
                                     als_v.at[s2(g)], sem_a.at[s2(g)])

    def cp_ald(g):
        return pltpu.make_async_copy(ald_f.at[didx.at[s2(g)]],
                                     ald_v.at[s2(g)], sem_a.at[s2(g)])

    def cp_rows(g):
        return pltpu.make_async_copy(xp.at[sd.at[si(g), 0]],
                                     rows_v.at[pl.ds(sr(g) * K, K)],
                                     sem_g.at[sr(g)])

    def cp_scat(g):
        return pltpu.make_async_copy(rows_v.at[pl.ds(sr(g) * K, K)],
                                     u_sh.at[sd.at[si(g), 1]],
                                     sem_s.at[si(g)])

    def cp_den(g):
        return pltpu.make_async_copy(ee_v.at[si(g)],
                                     den_sh.at[sd.at[si(g), 1]],
                                     sem_e.at[si(g)])

    def prep_idx(g):
        b = s2(g)
        for i in range(K // 16):
            sv = sd[si(g), 0, pl.ds(i * 16, 16)]
            dv = sd[si(g), 1, pl.ds(i * 16, 16)]
            sidx[b, pl.ds(i * 16, 16)] = sv * 128
            didx[b, pl.ds(i * 16, 16)] = dv * 128

    zero16 = jnp.zeros((16,), jnp.float32)
    zero32 = jnp.zeros((32,), jnp.bfloat16)
    for i in range(ZR):
        for p in range(D // 32):
            rows_v[i, pl.ds(p * 32, 32)] = zero32
    for i in range(RPT // 16 + 1):
        zden[pl.ds(i * 16, 16)] = zero16

    def zrow(g, carry):
        pltpu.sync_copy(rows_v.at[pl.ds(0, ZR)],
                        u_sh.at[pl.ds(s * RPT + g * ZR, ZR)])
        return carry
    lax.fori_loop(0, RPT // ZR, zrow, 0)

    @pl.when(s < NSUB - 1)
    def _():
        pltpu.sync_copy(zden.at[pl.ds(0, 640)],
                        den_sh.at[pl.ds(s * 640, 640)])

    @pl.when(s == NSUB - 1)
    def _():
        pltpu.sync_copy(zden.at[pl.ds(0, 400)],
                        den_sh.at[pl.ds((NSUB - 1) * 640, 400)])
    plsc.subcore_barrier()

    for k in range(4):
        cp_sd(k).start()
    for k in range(3):
        cp_sd(k).wait()
        prep_idx(k)
        cp_als(k).start()
        cp_ald(k).start()
        cp_rows(k).start()

    def chunk(g, carry):
        @pl.when(g >= 2)
        def _():
            cp_scat(g - 2).wait()
            cp_den(g - 2).wait()

        @pl.when(g + 4 < NCHUNK)
        def _():
            cp_sd(g + 4).start()

        @pl.when(g + 3 < NCHUNK)
        def _():
            cp_sd(g + 3).wait()
            prep_idx(g + 3)
            cp_als(g + 3).start()
            cp_ald(g + 3).start()
            cp_rows(g + 3).start()

        cp_als(g).wait()
        cp_ald(g).wait()
        cp_rows(g).wait()

        b2 = s2(g)
        bi = si(g)
        for i in range(K // 16):
            t = (als_v[b2, pl.ds(i * 16, 16)] + ald_v[b2, pl.ds(i * 16, 16)])
            e = jnp.where(t >= 0.0, t, 0.2 * t)
            ee_v[bi, pl.ds(i * 16, 16)] = jnp.exp(e)

        cp_den(g).start(add=True)

        base = sr(g) * K

        def srow(j4, cc):
            j = j4 * 4
            for u in range(4):
                eej = plsc.load_gather(
                    ee_v, [jnp.full((16,), bi, jnp.int32),
                           jnp.full((16,), j + u, jnp.int32)])
                eeb = plsc.pack(eej, eej, format=plsc.PackFormat.INTERLEAVED)
                r = base + j + u
                for p in range(D // 32):
                    rows_v[r, pl.ds(p * 32, 32)] = (
                        rows_v[r, pl.ds(p * 32, 32)] * eeb)
            return cc
        lax.fori_loop(0, K // 4, srow, 0)

        cp_scat(g).start(add=True)
        return carry
    lax.fori_loop(0, NCHUNK, chunk, 0)
    cp_scat(NCHUNK - 2).wait()
    cp_den(NCHUNK - 2).wait()
    cp_scat(NCHUNK - 1).wait()
    cp_den(NCHUNK - 1).wait()
    plsc.subcore_barrier()

    pltpu.sync_copy(u_sh.at[pl.ds(s * RPT, RPT)],
                    u_out.at[c, pl.ds(s * RPT, RPT)])

    @pl.when(s < NSUB - 1)
    def _():
        pltpu.sync_copy(den_sh.at[pl.ds(s * 640, 640)],
                        den_out.at[c, pl.ds(s * 640, 640)])

    @pl.when(s == NSUB - 1)
    def _():
        pltpu.sync_copy(den_sh.at[pl.ds((NSUB - 1) * 640, 400)],
                        den_out.at[c, pl.ds((NSUB - 1) * 640, 400)])


@functools.lru_cache(maxsize=1)
def _edge_kernel():
    return pl.kernel(
        _edge_body,
        out_type=(jax.ShapeDtypeStruct((NCORE, N, D), jnp.bfloat16),
                  jax.ShapeDtypeStruct((NCORE, N), jnp.float32)),
        mesh=plsc.VectorSubcoreMesh(core_axis_name="c", subcore_axis_name="s"),
        compiler_params=pltpu.CompilerParams(use_tc_tiling_on_sc=False,
                                             needs_layout_passes=False),
        scratch_types=[
            pltpu.VMEM((RING_I, 2, K), jnp.int32),
            pltpu.VMEM((RING_2, K), jnp.int32),
            pltpu.VMEM((RING_2, K), jnp.int32),
            pltpu.VMEM((RING_2, K), jnp.float32),
            pltpu.VMEM((RING_2, K), jnp.float32),
            pltpu.VMEM((RING_I, K), jnp.float32),
            pltpu.VMEM((RING_R * K, D), jnp.bfloat16),
            pltpu.VMEM((RPT + 15, ), jnp.float32),
            pltpu.VMEM_SHARED((N, D), jnp.bfloat16),
            pltpu.VMEM_SHARED((N,), jnp.float32),
            pltpu.SemaphoreType.DMA((RING_I,)),
            pltpu.SemaphoreType.DMA((RING_2,)),
            pltpu.SemaphoreType.DMA((RING_R,)),
            pltpu.SemaphoreType.DMA((RING_I,)),
            pltpu.SemaphoreType.DMA((RING_I,)),
        ],
    )


def _mid_body(u_ref, den_ref, skip_ref, b1_ref, g1_ref, bb1_ref,
              w2_ref, as_ref, ad_ref,
              xp_ref, als_ref, ald_ref, x1_ref):
    u = u_ref[0].astype(jnp.float32) + u_ref[1].astype(jnp.float32)
    gat = u / (den_ref[...] + 1e-16) + b1_ref[...]
    m = jnp.mean(gat, axis=-1, keepdims=True)
    v = jnp.mean((gat - m) ** 2, axis=-1, keepdims=True)
    ln = (gat - m) / jnp.sqrt(v + 1e-5) * g1_ref[...] + bb1_ref[...]
    elu = jnp.where(ln > 0, ln, jnp.exp(jnp.minimum(ln, 0.0)) - 1.0)
    x1 = elu + skip_ref[...]
    x1_ref[...] = x1
    xp = jnp.dot(x1, w2_ref[...], preferred_element_type=jnp.float32)
    als = lax.dot_general(xp, as_ref[...], (((1,), (1,)), ((), ())),
                          preferred_element_type=jnp.float32)
    ald = lax.dot_general(xp, ad_ref[...], (((1,), (1,)), ((), ())),
                          preferred_element_type=jnp.float32)
    xp_ref[...] = xp.astype(jnp.bfloat16)
    als_ref[...] = jnp.broadcast_to(als, (als.shape[0], D))
    ald_ref[...] = jnp.broadcast_to(ald, (ald.shape[0], D))


def _mid(u, den, skip, b1, ln1_g, ln1_b, W2, a_src, a_dst):
    return pl.pallas_call(
        _mid_body,
        grid=(NBLK,),
        in_specs=[
            pl.BlockSpec((2, BLK, D), lambda i: (0, i, 0)),
            pl.BlockSpec((BLK, 1), lambda i: (i, 0)),
            pl.BlockSpec((BLK, D), lambda i: (i, 0)),
            pl.BlockSpec((1, D), lambda i: (0, 0)),
            pl.BlockSpec((1, D), lambda i: (0, 0)),
            pl.BlockSpec((1, D), lambda i: (0, 0)),
            pl.BlockSpec((D, D), lambda i: (0, 0)),
            pl.BlockSpec((1, D), lambda i: (0, 0)),
            pl.BlockSpec((1, D), lambda i: (0, 0)),
        ],
        out_specs=[
            pl.BlockSpec((BLK, D), lambda i: (i, 0)),
            pl.BlockSpec((BLK, D), lambda i: (i, 0)),
            pl.BlockSpec((BLK, D), lambda i: (i, 0)),
            pl.BlockSpec((BLK, D), lambda i: (i, 0)),
        ],
        out_shape=[
            jax.ShapeDtypeStruct((N, D), jnp.bfloat16),
            jax.ShapeDtypeStruct((N, D), jnp.float32),
            jax.ShapeDtypeStruct((N, D), jnp.float32),
            jax.ShapeDtypeStruct((N, D), jnp.float32),
        ],
    )(u, den, skip, b1, ln1_g, ln1_b, W2, a_src, a_dst)


def _final_body(u_ref, den_ref, x1_ref, b2_ref, g2_ref, bb2_ref, batch_ref,
                wfc_ref, bfc_ref, bng_ref, bnb_ref,
                out_ref, acc_sum, acc_cnt):
    i = pl.program_id(0)
    u = u_ref[0].astype(jnp.float32) + u_ref[1].astype(jnp.float32)
    gat = u / (den_ref[...] + 1e-16) + b2_ref[...]
    pre = gat + x1_ref[...]
    m = jnp.mean(pre, axis=-1, keepdims=True)
    v = jnp.mean((pre - m) ** 2, axis=-1, keepdims=True)
    x2 = (pre - m) / jnp.sqrt(v + 1e-5) * g2_ref[...] + bb2_ref[...]
    emb = jnp.where(x2 > 0, x2, jnp.exp(jnp.minimum(x2, 0.0)) - 1.0)

    batch = batch_ref[...]
    gids = lax.broadcasted_iota(jnp.int32, (1, G), 1)
    mask = (batch == gids).astype(jnp.float32)
    part_sum = lax.dot_general(mask, emb, (((0,), (0,)), ((), ())),
                               preferred_element_type=jnp.float32)
    ones_blk = jnp.ones((emb.shape[0], D), jnp.float32)
    part_cnt = lax.dot_general(mask, ones_blk, (((0,), (0,)), ((), ())),
                               preferred_element_type=jnp.float32)

    @pl.when(i == 0)
    def _():
        acc_sum[...] = jnp.zeros_like(acc_sum)
        acc_cnt[...] = jnp.zeros_like(acc_cnt)

    acc_sum[...] += part_sum
    acc_cnt[...] += part_cnt

    @pl.when(i == NBLK - 1)
    def _():
        graph_emb = acc_sum[...] / jnp.maximum(acc_cnt[...], 1.0)
        logits = jnp.dot(graph_emb, wfc_ref[...],
                         preferred_element_type=jnp.float32) + bfc_ref[...]
        out_ref[...] = logits / jnp.sqrt(1.0 + 1e-5) * bng_ref[...] + bnb_ref[...]


def _final(u, den, x1, b2, ln2_g, ln2_b, batch, W_fc, b_fc, bn_g, bn_b):
    return pl.pallas_call(
        _final_body,
        grid=(NBLK,),
        in_specs=[
            pl.BlockSpec((2, BLK, D), lambda i: (0, i, 0)),
            pl.BlockSpec((BLK, 1), lambda i: (i, 0)),
            pl.BlockSpec((BLK, D), lambda i: (i, 0)),
            pl.BlockSpec((1, D), lambda i: (0, 0)),
            pl.BlockSpec((1, D), lambda i: (0, 0)),
            pl.BlockSpec((1, D), lambda i: (0, 0)),
            pl.BlockSpec((BLK, 1), lambda i: (i, 0)),
            pl.BlockSpec((D, OUT), lambda i: (0, 0)),
            pl.BlockSpec((1, OUT), lambda i: (0, 0)),
            pl.BlockSpec((1, OUT), lambda i: (0, 0)),
            pl.BlockSpec((1, OUT), lambda i: (0, 0)),
        ],
        out_specs=pl.BlockSpec((G, OUT), lambda i: (0, 0)),
        out_shape=jax.ShapeDtypeStruct((G, OUT), jnp.float32),
        scratch_shapes=[
            pltpu.VMEM((G, D), jnp.float32),
            pltpu.VMEM((G, D), jnp.float32),
        ],
    )(u, den, x1, b2, ln2_g, ln2_b, batch, W_fc, b_fc, bn_g, bn_b)


def kernel(x, edge_index, batch, W1, a1_src, a1_dst, b1, ln1_g, ln1_b,
           W_skip, b_skip, W2, a2_src, a2_dst, b2, ln2_g, ln2_b,
           W_fc, b_fc, bn_g, bn_b):
    src4 = edge_index[0].reshape(NCORE, NSUB, NCHUNK, K)
    dst4 = edge_index[1].reshape(NCORE, NSUB, NCHUNK, K)
    sd4 = jnp.stack([src4, dst4], axis=3)

    xp1, als1, ald1, skip = _prologue1(
        x, W1, a1_src, a1_dst, W_skip, b_skip.reshape(1, D))
    U1, den1 = _edge_kernel()(xp1, als1.reshape(N * D), ald1.reshape(N * D),
                              sd4)
    den1n = (den1[0] + den1[1]).reshape(N, 1)
    xp2, als2, ald2, x1 = _mid(
        U1, den1n, skip, b1.reshape(1, D), ln1_g.reshape(1, D),
        ln1_b.reshape(1, D), W2, a2_src, a2_dst)
    U2, den2 = _edge_kernel()(xp2, als2.reshape(N * D), ald2.reshape(N * D),
                              sd4)
    den2n = (den2[0] + den2[1]).reshape(N, 1)
    logits = _final(
        U2, den2n, x1, b2.reshape(1, D), ln2_g.reshape(1, D),
        ln2_b.reshape(1, D), batch.reshape(N, 1).astype(jnp.int32),
        W_fc, b_fc.reshape(1, OUT), bn_g.reshape(1, OUT), bn_b.reshape(1, OUT))
    return logits
